# Initial kernel scaffold; baseline (speedup 1.0000x reference)
import jax
import jax.numpy as jnp
from jax.experimental import pallas as pl


def kernel(x, edge_index, edge_vec, batch, params):
    raise NotImplementedError("write your pallas kernel here")


# speedup vs baseline: 1.5384x; 1.5384x over previous
import functools

import jax
import jax.numpy as jnp
import numpy as np
from jax import lax
from jax.experimental import pallas as pl
from jax.experimental.pallas import tpu as pltpu
from jax.experimental.pallas import tpu_sc as plsc

H = 64
N_NODES = 10000
N_EDGES = 320000
N_GRAPHS = 16
SQRT3 = float(np.sqrt(3.0))

NC = 2
NS = 16
NW = NC * NS

_SC_MESH = plsc.VectorSubcoreMesh(core_axis_name="c", subcore_axis_name="s")


def _sc_gather_pair(xh, row, col):
    E = row.shape[0]
    CH = 80
    iters = E // (NW * CH)

    @functools.partial(
        pl.kernel,
        mesh=_SC_MESH,
        out_type=[
            jax.ShapeDtypeStruct((E, H), jnp.float32),
            jax.ShapeDtypeStruct((E, H), jnp.float32),
        ],
        scratch_types=[
            pltpu.VMEM((CH,), jnp.int32),
            pltpu.VMEM((CH, H), jnp.float32),
            pltpu.SemaphoreType.DMA,
        ],
        compiler_params=pltpu.CompilerParams(use_tc_tiling_on_sc=False),
    )
    def k(xh_hbm, row_hbm, col_hbm, gr_hbm, gc_hbm, idx_v, buf_v, sem):
        wid = lax.axis_index("s") * NC + lax.axis_index("c")
        base = wid * (CH * iters)

        def body(j, _):
            off = base + j * CH
            pltpu.sync_copy(row_hbm.at[pl.ds(off, CH)], idx_v)
            pltpu.async_copy(xh_hbm.at[idx_v], buf_v, sem).wait()
            pltpu.sync_copy(buf_v, gr_hbm.at[pl.ds(off, CH)])
            pltpu.sync_copy(col_hbm.at[pl.ds(off, CH)], idx_v)
            pltpu.async_copy(xh_hbm.at[idx_v], buf_v, sem).wait()
            pltpu.sync_copy(buf_v, gc_hbm.at[pl.ds(off, CH)])
            return 0

        lax.fori_loop(0, iters, body, 0)

    return k(xh, row, col)


def _sc_scatter_add(vals, idx, zeros, nseg, ch):
    EP = vals.shape[0]
    iters = EP // (NW * ch)

    @functools.partial(
        pl.kernel,
        mesh=_SC_MESH,
        out_type=jax.ShapeDtypeStruct((2 * nseg, H), jnp.float32),
        scratch_types=[
            pltpu.VMEM((ch,), jnp.int32),
            pltpu.VMEM((ch, H), jnp.float32),
            pltpu.VMEM_SHARED((nseg, H), jnp.float32),
            pltpu.SemaphoreType.DMA,
        ],
        compiler_params=pltpu.CompilerParams(use_tc_tiling_on_sc=False),
    )
    def k(vals_hbm, idx_hbm, zeros_hbm, out_hbm, idx_v, val_v, acc, sem):
        cid = lax.axis_index("c")
        sid = lax.axis_index("s")
        wid = sid * NC + cid

        @pl.when(sid == 0)
        def _():
            pltpu.sync_copy(zeros_hbm, acc)

        plsc.subcore_barrier()
        base = wid * (ch * iters)

        def body(j, _):
            off = base + j * ch
            pltpu.sync_copy(idx_hbm.at[pl.ds(off, ch)], idx_v)
            pltpu.sync_copy(vals_hbm.at[pl.ds(off, ch)], val_v)
            pltpu.sync_copy(val_v, acc.at[idx_v], add=True)
            return 0

        lax.fori_loop(0, iters, body, 0)
        plsc.subcore_barrier()

        @pl.when(sid == 0)
        def _():
            pltpu.sync_copy(acc, out_hbm.at[pl.ds(cid * nseg, nseg)])

    return k(vals, idx, zeros)


def _sc_counts(idx, ones, zeros):
    E = idx.shape[0]
    CH = 80
    iters = E // (NW * CH)
    W = 16

    @functools.partial(
        pl.kernel,
        mesh=_SC_MESH,
        out_type=jax.ShapeDtypeStruct((2 * N_NODES, W), jnp.float32),
        scratch_types=[
            pltpu.VMEM((CH,), jnp.int32),
            pltpu.VMEM((CH, W), jnp.float32),
            pltpu.VMEM_SHARED((N_NODES, W), jnp.float32),
            pltpu.SemaphoreType.DMA,
        ],
        compiler_params=pltpu.CompilerParams(use_tc_tiling_on_sc=False),
    )
    def k(idx_hbm, ones_hbm, zeros_hbm, out_hbm, idx_v, ones_v, acc, sem):
        cid = lax.axis_index("c")
        sid = lax.axis_index("s")
        wid = sid * NC + cid
        pltpu.sync_copy(ones_hbm, ones_v)

        @pl.when(sid == 0)
        def _():
            pltpu.sync_copy(zeros_hbm, acc)

        plsc.subcore_barrier()
        base = wid * (CH * iters)

        def body(j, _):
            off = base + j * CH
            pltpu.sync_copy(idx_hbm.at[pl.ds(off, CH)], idx_v)
            pltpu.sync_copy(ones_v, acc.at[idx_v], add=True)
            return 0

        lax.fori_loop(0, iters, body, 0)
        plsc.subcore_barrier()

        @pl.when(sid == 0)
        def _():
            pltpu.sync_copy(acc, out_hbm.at[pl.ds(cid * N_NODES, N_NODES)])

    return k(idx, ones, zeros)


def _dot(a, b):
    return jax.lax.dot_general(a, b, (((1,), (0,)), ((), ())),
                               preferred_element_type=jnp.float32,
                               precision=jax.lax.Precision.HIGHEST)


def _dot_bf(a, b):
    return jax.lax.dot_general(a.astype(jnp.bfloat16), b.astype(jnp.bfloat16),
                               (((1,), (0,)), ((), ())),
                               preferred_element_type=jnp.float32)


def _ln_prelu(t, lnw, lnb, a):
    mu = jnp.mean(t, axis=-1, keepdims=True)
    var = jnp.mean((t - mu) ** 2, axis=-1, keepdims=True)
    tn = (t - mu) / jnp.sqrt(var + 1e-5) * lnw + lnb
    return jnp.where(tn >= 0, tn, a * tn)


def _tc_encoders(x, ev, ne1w, ne1b, ne_a, ne2w, ne2b,
                 ee1w, ee1b, ee_a, ee2w, ee2b):
    N, F = x.shape
    E = ev.shape[0]
    BN = 2000
    BE = 4000

    def node_k(x_ref, w1_ref, b1_ref, a_ref, w2_ref, b2_ref, out_ref):
        h = _dot_bf(x_ref[...], w1_ref[...]) + b1_ref[...]
        h = jnp.where(h >= 0, h, a_ref[...] * h)
        out_ref[...] = _dot_bf(h, w2_ref[...]) + b2_ref[...]

    xh = pl.pallas_call(
        node_k,
        grid=(N // BN,),
        in_specs=[
            pl.BlockSpec((BN, F), lambda i: (i, 0)),
            pl.BlockSpec((F, H), lambda i: (0, 0)),
            pl.BlockSpec((1, H), lambda i: (0, 0)),
            pl.BlockSpec((1, 1), lambda i: (0, 0)),
            pl.BlockSpec((H, H), lambda i: (0, 0)),
            pl.BlockSpec((1, H), lambda i: (0, 0)),
        ],
        out_specs=pl.BlockSpec((BN, H), lambda i: (i, 0)),
        out_shape=jax.ShapeDtypeStruct((N, H), jnp.float32),
        compiler_params=pltpu.CompilerParams(
            dimension_semantics=("parallel",)),
    )(x, ne1w, ne1b, ne_a, ne2w, ne2b)

    def edge_k(ev_ref, w1_ref, b1_ref, a_ref, w2_ref, b2_ref, out_ref):
        evb = ev_ref[...]
        vx = evb[:, 0:1]
        vy = evb[:, 1:2]
        vz = evb[:, 2:3]
        r = jnp.sqrt(vx * vx + vy * vy + vz * vz)
        u = 2.0 * (r * 0.25 - 1.0)
        cut = (1.0 - jnp.cos(jnp.pi * u)) * 0.5
        cut = jnp.where(u > 0.0, 0.0, cut)
        cut = jnp.where(u < -1.0, 1.0, cut)
        ea0 = jnp.concatenate(
            [cut, cut * (SQRT3 * (vy / r)), cut * (SQRT3 * (vz / r)),
             cut * (SQRT3 * (vx / r))], axis=1)
        h = _dot_bf(ea0, w1_ref[...]) + b1_ref[...]
        h = jnp.where(h >= 0, h, a_ref[...] * h)
        out_ref[...] = _dot_bf(h, w2_ref[...]) + b2_ref[...]

    ea = pl.pallas_call(
        edge_k,
        grid=(E // BE,),
        in_specs=[
            pl.BlockSpec((BE, 3), lambda i: (i, 0)),
            pl.BlockSpec((4, H), lambda i: (0, 0)),
            pl.BlockSpec((1, H), lambda i: (0, 0)),
            pl.BlockSpec((1, 1), lambda i: (0, 0)),
            pl.BlockSpec((H, H), lambda i: (0, 0)),
            pl.BlockSpec((1, H), lambda i: (0, 0)),
        ],
        out_specs=pl.BlockSpec((BE, H), lambda i: (i, 0)),
        out_shape=jax.ShapeDtypeStruct((E, H), jnp.float32),
        compiler_params=pltpu.CompilerParams(
            dimension_semantics=("parallel",)),
    )(ev, ee1w, ee1b, ee_a, ee2w, ee2b)

    return xh, ea


def _tc_edge_mlp(gr, gc, ea, e1w, e1b, elnw, elnb, e_a, e2w, e2b):
    E = ea.shape[0]
    BE = 2000

    def k(gr_ref, gc_ref, ea_ref, w1_ref, b1_ref, lnw_ref, lnb_ref, a_ref,
          w2_ref, b2_ref, new_ref, out_ref):
        w1 = w1_ref[...]
        eab = ea_ref[...]
        t = (_dot_bf(gr_ref[...], w1[0:H, :])
             + _dot_bf(gc_ref[...], w1[H:2 * H, :])
             + _dot_bf(eab, w1[2 * H:3 * H, :])
             + b1_ref[...])
        tp = _ln_prelu(t, lnw_ref[...], lnb_ref[...], a_ref[...])
        ea_new = _dot_bf(tp, w2_ref[...]) + b2_ref[...]
        new_ref[...] = ea_new
        out_ref[...] = eab + ea_new

    return pl.pallas_call(
        k,
        grid=(E // BE,),
        in_specs=[
            pl.BlockSpec((BE, H), lambda i: (i, 0)),
            pl.BlockSpec((BE, H), lambda i: (i, 0)),
            pl.BlockSpec((BE, H), lambda i: (i, 0)),
            pl.BlockSpec((3 * H, 2 * H), lambda i: (0, 0)),
            pl.BlockSpec((1, 2 * H), lambda i: (0, 0)),
            pl.BlockSpec((1, 2 * H), lambda i: (0, 0)),
            pl.BlockSpec((1, 2 * H), lambda i: (0, 0)),
            pl.BlockSpec((1, 1), lambda i: (0, 0)),
            pl.BlockSpec((2 * H, H), lambda i: (0, 0)),
            pl.BlockSpec((1, H), lambda i: (0, 0)),
        ],
        out_specs=[
            pl.BlockSpec((BE, H), lambda i: (i, 0)),
            pl.BlockSpec((BE, H), lambda i: (i, 0)),
        ],
        out_shape=[
            jax.ShapeDtypeStruct((E, H), jnp.float32),
            jax.ShapeDtypeStruct((E, H), jnp.float32),
        ],
        compiler_params=pltpu.CompilerParams(
            dimension_semantics=("parallel",)),
    )(gr, gc, ea, e1w, e1b, elnw, elnb, e_a, e2w, e2b)


def _tc_node_mlp(xh, sums2, cnt2, n1w, n1b, nlnw, nlnb, n_a, n2w, n2b):
    N = xh.shape[0]
    BN = 2000

    def k(xh_ref, s0_ref, s1_ref, c0_ref, c1_ref, w1_ref, b1_ref,
          lnw_ref, lnb_ref, a_ref, w2_ref, b2_ref, out_ref):
        cnt = c0_ref[...][:, 0:1] + c1_ref[...][:, 0:1]
        agg = (s0_ref[...] + s1_ref[...]) / jnp.maximum(cnt, 1.0)
        xhb = xh_ref[...]
        w1 = w1_ref[...]
        t = (_dot_bf(xhb, w1[0:H, :]) + _dot_bf(agg, w1[H:2 * H, :])
             + b1_ref[...])
        tp = _ln_prelu(t, lnw_ref[...], lnb_ref[...], a_ref[...])
        out_ref[...] = xhb + _dot_bf(tp, w2_ref[...]) + b2_ref[...]

    NB = N // BN
    return pl.pallas_call(
        k,
        grid=(NB,),
        in_specs=[
            pl.BlockSpec((BN, H), lambda i: (i, 0)),
            pl.BlockSpec((BN, H), lambda i: (i, 0)),
            pl.BlockSpec((BN, H), lambda i: (i + NB, 0)),
            pl.BlockSpec((BN, 16), lambda i: (i, 0)),
            pl.BlockSpec((BN, 16), lambda i: (i + NB, 0)),
            pl.BlockSpec((2 * H, 2 * H), lambda i: (0, 0)),
            pl.BlockSpec((1, 2 * H), lambda i: (0, 0)),
            pl.BlockSpec((1, 2 * H), lambda i: (0, 0)),
            pl.BlockSpec((1, 2 * H), lambda i: (0, 0)),
            pl.BlockSpec((1, 1), lambda i: (0, 0)),
            pl.BlockSpec((2 * H, H), lambda i: (0, 0)),
            pl.BlockSpec((1, H), lambda i: (0, 0)),
        ],
        out_specs=pl.BlockSpec((BN, H), lambda i: (i, 0)),
        out_shape=jax.ShapeDtypeStruct((N, H), jnp.float32),
        compiler_params=pltpu.CompilerParams(
            dimension_semantics=("parallel",)),
    )(xh, sums2, sums2, cnt2, cnt2, n1w, n1b, nlnw, nlnb, n_a, n2w, n2b)


def _tc_decoder(xh, batch_r, emb, decw, decb, o1w, o1b, o2w_r, o2b):
    N = xh.shape[0]
    BN = 2000
    NB = N // BN
    B = N_GRAPHS

    def k(xh_ref, b_ref, emb_ref, decw_ref, decb_ref, o1w_ref, o1b_ref,
          o2w_ref, o2b_ref, out_ref, acc_ref):
        i = pl.program_id(0)
        bb = b_ref[...].reshape(1, BN)
        gid = jax.lax.broadcasted_iota(jnp.int32, (B, 1), 0)
        oh = (bb == gid).astype(jnp.float32)
        part = _dot(oh, xh_ref[...])

        @pl.when(i == 0)
        def _():
            acc_ref[...] = jnp.zeros_like(acc_ref)

        acc_ref[...] += part

        @pl.when(i == NB - 1)
        def _():
            gsum = acc_ref[...]
            graph = _dot_bf(gsum, decw_ref[...]) + decb_ref[...]
            o1w = o1w_ref[...]
            A = _dot_bf(emb_ref[...], o1w[0:H, :])
            Bm = _dot_bf(graph, o1w[H:2 * H, :])
            h = Bm[:, None, :] + A[None, :, :] + o1b_ref[...][None, :, :]
            h = jnp.where(h >= 0, h, 0.01 * h)
            hb = h.astype(jnp.bfloat16).astype(jnp.float32)
            ow = o2w_ref[...].astype(jnp.bfloat16).astype(jnp.float32)
            dos = jnp.sum(hb * ow[None, :, :], axis=-1) + o2b_ref[...]
            out_ref[...] = dos

    return pl.pallas_call(
        k,
        grid=(NB,),
        in_specs=[
            pl.BlockSpec((BN, H), lambda i: (i, 0)),
            pl.BlockSpec((1, 1, BN), lambda i: (i, 0, 0)),
            pl.BlockSpec((51, H), lambda i: (0, 0)),
            pl.BlockSpec((H, H), lambda i: (0, 0)),
            pl.BlockSpec((1, H), lambda i: (0, 0)),
            pl.BlockSpec((2 * H, H), lambda i: (0, 0)),
            pl.BlockSpec((1, H), lambda i: (0, 0)),
            pl.BlockSpec((1, H), lambda i: (0, 0)),
            pl.BlockSpec((1, 1), lambda i: (0, 0)),
        ],
        out_specs=pl.BlockSpec((B, 51), lambda i: (0, 0)),
        out_shape=jax.ShapeDtypeStruct((B, 51), jnp.float32),
        scratch_shapes=[pltpu.VMEM((B, H), jnp.float32)],
        compiler_params=pltpu.CompilerParams(
            dimension_semantics=("arbitrary",)),
    )(xh, batch_r, emb, decw, decb, o1w, o1b, o2w_r, o2b)


def _r2(v):
    return jnp.reshape(v, (1, -1))


def kernel(x, edge_index, edge_vec, batch, params):
    N = x.shape[0]
    E = edge_vec.shape[0]
    row = edge_index[0]
    col = edge_index[1]

    zeros_nh = jnp.zeros((N, H), jnp.float32)
    zeros_n16 = jnp.zeros((N, 16), jnp.float32)
    ones_ch16 = jnp.ones((80, 16), jnp.float32)

    p = params
    xh, ea = _tc_encoders(
        x, edge_vec,
        p["ne1"]["w"], _r2(p["ne1"]["b"]), jnp.reshape(p["ne_a"], (1, 1)),
        p["ne2"]["w"], _r2(p["ne2"]["b"]),
        p["ee1"]["w"], _r2(p["ee1"]["b"]), jnp.reshape(p["ee_a"], (1, 1)),
        p["ee2"]["w"], _r2(p["ee2"]["b"]))

    cnt2 = _sc_counts(col, ones_ch16, zeros_n16)

    for lp in p["layers"]:
        gr, gc = _sc_gather_pair(xh, row, col)
        ea_new, ea = _tc_edge_mlp(
            gr, gc, ea, lp["e1"]["w"], _r2(lp["e1"]["b"]),
            _r2(lp["eln_w"]), _r2(lp["eln_b"]),
            jnp.reshape(lp["e_a"], (1, 1)), lp["e2"]["w"], _r2(lp["e2"]["b"]))
        sums2 = _sc_scatter_add(ea_new, col, zeros_nh, N, 80)
        xh = _tc_node_mlp(
            xh, sums2, cnt2, lp["n1"]["w"], _r2(lp["n1"]["b"]),
            _r2(lp["nln_w"]), _r2(lp["nln_b"]),
            jnp.reshape(lp["n_a"], (1, 1)), lp["n2"]["w"], _r2(lp["n2"]["b"]))

    batch_r = jnp.reshape(batch.astype(jnp.int32), (N // 2000, 1, 2000))
    out = _tc_decoder(
        xh, batch_r, p["emb"], p["dec"]["w"], _r2(p["dec"]["b"]),
        p["o1"]["w"], _r2(p["o1"]["b"]),
        jnp.reshape(p["o2"]["w"], (1, H)), jnp.reshape(p["o2"]["b"], (1, 1)))
    return out


# speedup vs baseline: 2.9479x; 1.9162x over previous
import functools

import jax
import jax.numpy as jnp
import numpy as np
from jax import lax
from jax.experimental import pallas as pl
from jax.experimental.pallas import tpu as pltpu
from jax.experimental.pallas import tpu_sc as plsc

H = 64
N_NODES = 10000
N_EDGES = 320000
N_GRAPHS = 16
SQRT3 = float(np.sqrt(3.0))

NC = 2
NS = 16
NW = NC * NS

_SC_MESH = plsc.VectorSubcoreMesh(core_axis_name="c", subcore_axis_name="s")


K_CH = 100
CH = 100
G = 10


def _sc_gather_pair(xh_bf, row2, col2):
    E = NW * K_CH * CH

    @functools.partial(
        pl.kernel,
        mesh=_SC_MESH,
        out_type=[
            jax.ShapeDtypeStruct((E, H), jnp.bfloat16),
            jax.ShapeDtypeStruct((E, H), jnp.bfloat16),
        ],
        scratch_types=[
            pltpu.VMEM((K_CH, CH), jnp.int32),
            pltpu.VMEM((K_CH, CH), jnp.int32),
            pltpu.VMEM((G * CH, H), jnp.bfloat16),
            pltpu.VMEM((G * CH, H), jnp.bfloat16),
            pltpu.SemaphoreType.DMA,
            pltpu.SemaphoreType.DMA,
        ],
        compiler_params=pltpu.CompilerParams(use_tc_tiling_on_sc=False),
    )
    def k(xh_hbm, row_hbm, col_hbm, gr_hbm, gc_hbm,
          idx_r, idx_c, buf_r, buf_c, sem_r, sem_c):
        wid = lax.axis_index("s") * NC + lax.axis_index("c")
        pltpu.sync_copy(row_hbm.at[wid], idx_r)
        pltpu.sync_copy(col_hbm.at[wid], idx_c)
        base = wid * (K_CH * CH)

        def group(o, _):
            waits = []
            for g in range(G):
                j = o * G + g
                waits.append(pltpu.async_copy(
                    xh_hbm.at[idx_r.at[j]], buf_r.at[pl.ds(g * CH, CH)],
                    sem_r))
                waits.append(pltpu.async_copy(
                    xh_hbm.at[idx_c.at[j]], buf_c.at[pl.ds(g * CH, CH)],
                    sem_c))
            for w in waits:
                w.wait()
            off = base + o * (G * CH)
            pltpu.sync_copy(buf_r, gr_hbm.at[pl.ds(off, G * CH)])
            pltpu.sync_copy(buf_c, gc_hbm.at[pl.ds(off, G * CH)])
            return 0

        lax.fori_loop(0, K_CH // G, group, 0)

    return k(xh_bf, row2, col2)


def _sc_scatter_add(vals, idx2, zeros, nseg):
    E = vals.shape[0]

    @functools.partial(
        pl.kernel,
        mesh=_SC_MESH,
        out_type=jax.ShapeDtypeStruct((2 * nseg, H), jnp.float32),
        scratch_types=[
            pltpu.VMEM((K_CH, CH), jnp.int32),
            pltpu.VMEM((G * CH, H), jnp.float32),
            pltpu.VMEM_SHARED((nseg, H), jnp.float32),
            pltpu.SemaphoreType.DMA,
            pltpu.SemaphoreType.DMA,
        ],
        compiler_params=pltpu.CompilerParams(use_tc_tiling_on_sc=False),
    )
    def k(vals_hbm, idx_hbm, zeros_hbm, out_hbm, idx_v, val_v, acc,
          sem_l, sem_s):
        cid = lax.axis_index("c")
        sid = lax.axis_index("s")
        wid = sid * NC + cid
        pltpu.sync_copy(idx_hbm.at[wid], idx_v)

        @pl.when(sid == 0)
        def _():
            pltpu.sync_copy(zeros_hbm, acc)

        plsc.subcore_barrier()
        base = wid * (K_CH * CH)

        def group(o, _):
            off = base + o * (G * CH)
            loads = [
                pltpu.async_copy(vals_hbm.at[pl.ds(off + g * CH, CH)],
                                 val_v.at[pl.ds(g * CH, CH)], sem_l)
                for g in range(G)
            ]
            for w in loads:
                w.wait()
            adds = [
                pltpu.async_copy(val_v.at[pl.ds(g * CH, CH)],
                                 acc.at[idx_v.at[o * G + g]], sem_s,
                                 add=True)
                for g in range(G)
            ]
            for w in adds:
                w.wait()
            return 0

        lax.fori_loop(0, K_CH // G, group, 0)
        plsc.subcore_barrier()

        @pl.when(sid == 0)
        def _():
            pltpu.sync_copy(acc, out_hbm.at[pl.ds(cid * nseg, nseg)])

    return k(vals, idx2, zeros)


def _sc_counts(idx2, ones, zeros):
    W = 16

    @functools.partial(
        pl.kernel,
        mesh=_SC_MESH,
        out_type=jax.ShapeDtypeStruct((2 * N_NODES, W), jnp.float32),
        scratch_types=[
            pltpu.VMEM((K_CH, CH), jnp.int32),
            pltpu.VMEM((CH, W), jnp.float32),
            pltpu.VMEM_SHARED((N_NODES, W), jnp.float32),
            pltpu.SemaphoreType.DMA,
        ],
        compiler_params=pltpu.CompilerParams(use_tc_tiling_on_sc=False),
    )
    def k(idx_hbm, ones_hbm, zeros_hbm, out_hbm, idx_v, ones_v, acc, sem):
        cid = lax.axis_index("c")
        sid = lax.axis_index("s")
        wid = sid * NC + cid
        pltpu.sync_copy(ones_hbm, ones_v)
        pltpu.sync_copy(idx_hbm.at[wid], idx_v)

        @pl.when(sid == 0)
        def _():
            pltpu.sync_copy(zeros_hbm, acc)

        plsc.subcore_barrier()

        def group(o, _):
            adds = [
                pltpu.async_copy(ones_v, acc.at[idx_v.at[o * G + g]], sem,
                                 add=True)
                for g in range(G)
            ]
            for w in adds:
                w.wait()
            return 0

        lax.fori_loop(0, K_CH // G, group, 0)
        plsc.subcore_barrier()

        @pl.when(sid == 0)
        def _():
            pltpu.sync_copy(acc, out_hbm.at[pl.ds(cid * N_NODES, N_NODES)])

    return k(idx2, ones, zeros)


def _dot(a, b):
    return jax.lax.dot_general(a, b, (((1,), (0,)), ((), ())),
                               preferred_element_type=jnp.float32,
                               precision=jax.lax.Precision.HIGHEST)


def _dot_bf(a, b):
    return jax.lax.dot_general(a.astype(jnp.bfloat16), b.astype(jnp.bfloat16),
                               (((1,), (0,)), ((), ())),
                               preferred_element_type=jnp.float32)


def _ln_prelu(t, lnw, lnb, a):
    mu = jnp.mean(t, axis=-1, keepdims=True)
    var = jnp.mean((t - mu) ** 2, axis=-1, keepdims=True)
    tn = (t - mu) / jnp.sqrt(var + 1e-5) * lnw + lnb
    return jnp.where(tn >= 0, tn, a * tn)


def _tc_encoders(x, ev, ne1w, ne1b, ne_a, ne2w, ne2b,
                 ee1w, ee1b, ee_a, ee2w, ee2b):
    N, F = x.shape
    E = ev.shape[1]
    BN = 2000
    BE = 3200

    def node_k(x_ref, w1_ref, b1_ref, a_ref, w2_ref, b2_ref,
               out_ref, outb_ref):
        h = _dot_bf(x_ref[...], w1_ref[...]) + b1_ref[...]
        h = jnp.where(h >= 0, h, a_ref[...] * h)
        xh = _dot_bf(h, w2_ref[...]) + b2_ref[...]
        out_ref[...] = xh
        outb_ref[...] = xh.astype(jnp.bfloat16)

    xh, xh_bf = pl.pallas_call(
        node_k,
        grid=(N // BN,),
        in_specs=[
            pl.BlockSpec((BN, F), lambda i: (i, 0)),
            pl.BlockSpec((F, H), lambda i: (0, 0)),
            pl.BlockSpec((1, H), lambda i: (0, 0)),
            pl.BlockSpec((1, 1), lambda i: (0, 0)),
            pl.BlockSpec((H, H), lambda i: (0, 0)),
            pl.BlockSpec((1, H), lambda i: (0, 0)),
        ],
        out_specs=[
            pl.BlockSpec((BN, H), lambda i: (i, 0)),
            pl.BlockSpec((BN, H), lambda i: (i, 0)),
        ],
        out_shape=[
            jax.ShapeDtypeStruct((N, H), jnp.float32),
            jax.ShapeDtypeStruct((N, H), jnp.bfloat16),
        ],
        compiler_params=pltpu.CompilerParams(
            dimension_semantics=("parallel",)),
    )(x, ne1w, ne1b, ne_a, ne2w, ne2b)

    def edge_k(ev_ref, w1_ref, b1_ref, a_ref, w2_ref, b2_ref, out_ref):
        evb = ev_ref[...]
        vx = evb[0:1, :]
        vy = evb[1:2, :]
        vz = evb[2:3, :]
        r = jnp.sqrt(vx * vx + vy * vy + vz * vz)
        u = 2.0 * (r * 0.25 - 1.0)
        cut = (1.0 - jnp.cos(jnp.pi * u)) * 0.5
        cut = jnp.where(u > 0.0, 0.0, cut)
        cut = jnp.where(u < -1.0, 1.0, cut)
        ea0t = jnp.concatenate(
            [cut, cut * (SQRT3 * (vy / r)), cut * (SQRT3 * (vz / r)),
             cut * (SQRT3 * (vx / r))], axis=0)
        h = jax.lax.dot_general(
            ea0t.astype(jnp.bfloat16), w1_ref[...].astype(jnp.bfloat16),
            (((0,), (0,)), ((), ())),
            preferred_element_type=jnp.float32) + b1_ref[...]
        h = jnp.where(h >= 0, h, a_ref[...] * h)
        out_ref[...] = _dot_bf(h, w2_ref[...]) + b2_ref[...]

    ea = pl.pallas_call(
        edge_k,
        grid=(E // BE,),
        in_specs=[
            pl.BlockSpec((3, BE), lambda i: (0, i)),
            pl.BlockSpec((4, H), lambda i: (0, 0)),
            pl.BlockSpec((1, H), lambda i: (0, 0)),
            pl.BlockSpec((1, 1), lambda i: (0, 0)),
            pl.BlockSpec((H, H), lambda i: (0, 0)),
            pl.BlockSpec((1, H), lambda i: (0, 0)),
        ],
        out_specs=pl.BlockSpec((BE, H), lambda i: (i, 0)),
        out_shape=jax.ShapeDtypeStruct((E, H), jnp.float32),
        compiler_params=pltpu.CompilerParams(
            dimension_semantics=("parallel",)),
    )(ev, ee1w, ee1b, ee_a, ee2w, ee2b)

    return xh, xh_bf, jnp.reshape(ea, (E // 2, 2 * H))


def _tc_edge_mlp(gr2, gc2, ea2, w1p, b1p, lnwp, lnbp, e_a, w2p, b2p):
    E2 = ea2.shape[0]
    BE2 = 2000
    HP = 2 * H
    D = 4 * H

    def ln(v, w, b):
        mu = jnp.mean(v, axis=-1, keepdims=True)
        var = jnp.mean((v - mu) ** 2, axis=-1, keepdims=True)
        return (v - mu) / jnp.sqrt(var + 1e-5) * w + b

    def k(gr_ref, gc_ref, ea_ref, w1_ref, b1_ref, lnw_ref, lnb_ref, a_ref,
          w2_ref, b2_ref, new_ref, out_ref):
        w1 = w1_ref[...]
        eab = ea_ref[...]
        t = (_dot_bf(gr_ref[...], w1[0:HP, :])
             + _dot_bf(gc_ref[...], w1[HP:2 * HP, :])
             + _dot_bf(eab, w1[2 * HP:3 * HP, :])
             + b1_ref[...])
        lnw = lnw_ref[...]
        lnb = lnb_ref[...]
        tn = jnp.concatenate(
            [ln(t[:, :HP], lnw[:, :HP], lnb[:, :HP]),
             ln(t[:, HP:], lnw[:, HP:], lnb[:, HP:])], axis=1)
        tp = jnp.where(tn >= 0, tn, a_ref[...] * tn)
        ea_new = _dot_bf(tp, w2_ref[...]) + b2_ref[...]
        new_ref[...] = ea_new
        out_ref[...] = eab + ea_new

    return pl.pallas_call(
        k,
        grid=(E2 // BE2,),
        in_specs=[
            pl.BlockSpec((BE2, HP), lambda i: (i, 0)),
            pl.BlockSpec((BE2, HP), lambda i: (i, 0)),
            pl.BlockSpec((BE2, HP), lambda i: (i, 0)),
            pl.BlockSpec((3 * HP, D), lambda i: (0, 0)),
            pl.BlockSpec((1, D), lambda i: (0, 0)),
            pl.BlockSpec((1, D), lambda i: (0, 0)),
            pl.BlockSpec((1, D), lambda i: (0, 0)),
            pl.BlockSpec((1, 1), lambda i: (0, 0)),
            pl.BlockSpec((D, HP), lambda i: (0, 0)),
            pl.BlockSpec((1, HP), lambda i: (0, 0)),
        ],
        out_specs=[
            pl.BlockSpec((BE2, HP), lambda i: (i, 0)),
            pl.BlockSpec((BE2, HP), lambda i: (i, 0)),
        ],
        out_shape=[
            jax.ShapeDtypeStruct((E2, HP), jnp.float32),
            jax.ShapeDtypeStruct((E2, HP), jnp.float32),
        ],
        compiler_params=pltpu.CompilerParams(
            dimension_semantics=("parallel",)),
    )(gr2, gc2, ea2, w1p, b1p, lnwp, lnbp, e_a, w2p, b2p)


def _tc_node_mlp(xh, sums2, cnt2, n1w, n1b, nlnw, nlnb, n_a, n2w, n2b):
    N = xh.shape[0]
    BN = 2000

    def k(xh_ref, s0_ref, s1_ref, c0_ref, c1_ref, w1_ref, b1_ref,
          lnw_ref, lnb_ref, a_ref, w2_ref, b2_ref, out_ref, outb_ref):
        cnt = c0_ref[...][:, 0:1] + c1_ref[...][:, 0:1]
        agg = (s0_ref[...] + s1_ref[...]) / jnp.maximum(cnt, 1.0)
        xhb = xh_ref[...]
        w1 = w1_ref[...]
        t = (_dot_bf(xhb, w1[0:H, :]) + _dot_bf(agg, w1[H:2 * H, :])
             + b1_ref[...])
        tp = _ln_prelu(t, lnw_ref[...], lnb_ref[...], a_ref[...])
        xh_out = xhb + _dot_bf(tp, w2_ref[...]) + b2_ref[...]
        out_ref[...] = xh_out
        outb_ref[...] = xh_out.astype(jnp.bfloat16)

    NB = N // BN
    return pl.pallas_call(
        k,
        grid=(NB,),
        in_specs=[
            pl.BlockSpec((BN, H), lambda i: (i, 0)),
            pl.BlockSpec((BN, H), lambda i: (i, 0)),
            pl.BlockSpec((BN, H), lambda i: (i + NB, 0)),
            pl.BlockSpec((BN, 16), lambda i: (i, 0)),
            pl.BlockSpec((BN, 16), lambda i: (i + NB, 0)),
            pl.BlockSpec((2 * H, 2 * H), lambda i: (0, 0)),
            pl.BlockSpec((1, 2 * H), lambda i: (0, 0)),
            pl.BlockSpec((1, 2 * H), lambda i: (0, 0)),
            pl.BlockSpec((1, 2 * H), lambda i: (0, 0)),
            pl.BlockSpec((1, 1), lambda i: (0, 0)),
            pl.BlockSpec((2 * H, H), lambda i: (0, 0)),
            pl.BlockSpec((1, H), lambda i: (0, 0)),
        ],
        out_specs=[
            pl.BlockSpec((BN, H), lambda i: (i, 0)),
            pl.BlockSpec((BN, H), lambda i: (i, 0)),
        ],
        out_shape=[
            jax.ShapeDtypeStruct((N, H), jnp.float32),
            jax.ShapeDtypeStruct((N, H), jnp.bfloat16),
        ],
        compiler_params=pltpu.CompilerParams(
            dimension_semantics=("parallel",)),
    )(xh, sums2, sums2, cnt2, cnt2, n1w, n1b, nlnw, nlnb, n_a, n2w, n2b)


def _tc_decoder(xh, batch_r, emb, decw, decb, o1w, o1b, o2w_r, o2b):
    N = xh.shape[0]
    BN = 2000
    NB = N // BN
    B = N_GRAPHS

    def k(xh_ref, b_ref, emb_ref, decw_ref, decb_ref, o1w_ref, o1b_ref,
          o2w_ref, o2b_ref, out_ref, acc_ref):
        i = pl.program_id(0)
        bb = b_ref[...].reshape(1, BN)
        gid = jax.lax.broadcasted_iota(jnp.int32, (B, 1), 0)
        oh = (bb == gid).astype(jnp.float32)
        part = _dot(oh, xh_ref[...])

        @pl.when(i == 0)
        def _():
            acc_ref[...] = jnp.zeros_like(acc_ref)

        acc_ref[...] += part

        @pl.when(i == NB - 1)
        def _():
            gsum = acc_ref[...]
            graph = _dot_bf(gsum, decw_ref[...]) + decb_ref[...]
            o1w = o1w_ref[...]
            A = _dot_bf(emb_ref[...], o1w[0:H, :])
            Bm = _dot_bf(graph, o1w[H:2 * H, :])
            h = Bm[:, None, :] + A[None, :, :] + o1b_ref[...][None, :, :]
            h = jnp.where(h >= 0, h, 0.01 * h)
            hb = h.astype(jnp.bfloat16).astype(jnp.float32)
            ow = o2w_ref[...].astype(jnp.bfloat16).astype(jnp.float32)
            dos = jnp.sum(hb * ow[None, :, :], axis=-1) + o2b_ref[...]
            out_ref[...] = dos

    return pl.pallas_call(
        k,
        grid=(NB,),
        in_specs=[
            pl.BlockSpec((BN, H), lambda i: (i, 0)),
            pl.BlockSpec((1, 1, BN), lambda i: (i, 0, 0)),
            pl.BlockSpec((51, H), lambda i: (0, 0)),
            pl.BlockSpec((H, H), lambda i: (0, 0)),
            pl.BlockSpec((1, H), lambda i: (0, 0)),
            pl.BlockSpec((2 * H, H), lambda i: (0, 0)),
            pl.BlockSpec((1, H), lambda i: (0, 0)),
            pl.BlockSpec((1, H), lambda i: (0, 0)),
            pl.BlockSpec((1, 1), lambda i: (0, 0)),
        ],
        out_specs=pl.BlockSpec((B, 51), lambda i: (0, 0)),
        out_shape=jax.ShapeDtypeStruct((B, 51), jnp.float32),
        scratch_shapes=[pltpu.VMEM((B, H), jnp.float32)],
        compiler_params=pltpu.CompilerParams(
            dimension_semantics=("arbitrary",)),
    )(xh, batch_r, emb, decw, decb, o1w, o1b, o2w_r, o2b)


def _r2(v):
    return jnp.reshape(v, (1, -1))


def kernel(x, edge_index, edge_vec, batch, params):
    N = x.shape[0]
    E = edge_vec.shape[0]
    row = edge_index[0]
    col = edge_index[1]

    zeros_nh = jnp.zeros((N, H), jnp.float32)
    zeros_n16 = jnp.zeros((N, 16), jnp.float32)
    ones_ch16 = jnp.ones((CH, 16), jnp.float32)
    row2 = jnp.reshape(row.astype(jnp.int32), (NW, K_CH, CH))
    col2 = jnp.reshape(col.astype(jnp.int32), (NW, K_CH, CH))

    p = params
    xh, xh_bf, ea = _tc_encoders(
        x, edge_vec.T,
        p["ne1"]["w"], _r2(p["ne1"]["b"]), jnp.reshape(p["ne_a"], (1, 1)),
        p["ne2"]["w"], _r2(p["ne2"]["b"]),
        p["ee1"]["w"], _r2(p["ee1"]["b"]), jnp.reshape(p["ee_a"], (1, 1)),
        p["ee2"]["w"], _r2(p["ee2"]["b"]))

    cnt2 = _sc_counts(col2, ones_ch16, zeros_n16)

    eye2 = jnp.eye(2, dtype=jnp.float32)

    def _pair_w(w):
        return jnp.kron(eye2, w)

    def _pair_b(b):
        return jnp.concatenate([_r2(b), _r2(b)], axis=1)

    for lp in p["layers"]:
        e1w = lp["e1"]["w"]
        w1p = jnp.concatenate(
            [_pair_w(e1w[0:H]), _pair_w(e1w[H:2 * H]),
             _pair_w(e1w[2 * H:3 * H])], axis=0)
        gr, gc = _sc_gather_pair(xh_bf, row2, col2)
        gr2 = jnp.reshape(gr, (E // 2, 2 * H))
        gc2 = jnp.reshape(gc, (E // 2, 2 * H))
        ea_new2, ea = _tc_edge_mlp(
            gr2, gc2, ea, w1p, _pair_b(lp["e1"]["b"]),
            _pair_b(lp["eln_w"]), _pair_b(lp["eln_b"]),
            jnp.reshape(lp["e_a"], (1, 1)), _pair_w(lp["e2"]["w"]),
            _pair_b(lp["e2"]["b"]))
        sums2 = _sc_scatter_add(jnp.reshape(ea_new2, (E, H)), col2, zeros_nh, N)
        xh, xh_bf = _tc_node_mlp(
            xh, sums2, cnt2, lp["n1"]["w"], _r2(lp["n1"]["b"]),
            _r2(lp["nln_w"]), _r2(lp["nln_b"]),
            jnp.reshape(lp["n_a"], (1, 1)), lp["n2"]["w"], _r2(lp["n2"]["b"]))

    batch_r = jnp.reshape(batch.astype(jnp.int32), (N // 2000, 1, 2000))
    out = _tc_decoder(
        xh, batch_r, p["emb"], p["dec"]["w"], _r2(p["dec"]["b"]),
        p["o1"]["w"], _r2(p["o1"]["b"]),
        jnp.reshape(p["o2"]["w"], (1, H)), jnp.reshape(p["o2"]["b"], (1, 1)))
    return out
